# baseline (speedup 1.0000x reference)
import functools

import jax
import jax.numpy as jnp
from jax import lax
from jax.experimental import pallas as pl
from jax.experimental.pallas import tpu as pltpu
from jax.experimental.pallas import tpu_sc as plsc

_CHUNK = 128


@functools.lru_cache(maxsize=None)
def _build(batch, emb, n_cores, n_subcores):
    n_workers = n_cores * n_subcores
    b_per_w = batch // n_workers
    n_chunks = b_per_w // _CHUNK

    mesh = plsc.VectorSubcoreMesh(
        core_axis_name="c",
        subcore_axis_name="s",
        num_cores=n_cores,
        num_subcores=n_subcores,
    )

    @functools.partial(
        pl.kernel,
        mesh=mesh,
        out_type=jax.ShapeDtypeStruct(
            (n_workers, n_chunks, _CHUNK, emb), jnp.bfloat16
        ),
        compiler_params=pltpu.CompilerParams(use_tc_tiling_on_sc=False),
        scratch_types=[
            pltpu.VMEM((n_chunks, _CHUNK), jnp.int32),
            pltpu.VMEM((n_chunks, _CHUNK, emb), jnp.bfloat16),
            pltpu.SemaphoreType.DMA,
        ],
    )
    def gather_kernel(id_hbm, tab_hbm, out_hbm, idx_v, rows_v, sem):
        wid = lax.axis_index("s") * n_cores + lax.axis_index("c")

        pltpu.sync_copy(id_hbm.at[wid], idx_v)

        copies = []
        for c in range(n_chunks):
            copies.append(
                pltpu.async_copy(
                    tab_hbm.at[idx_v.at[c]], rows_v.at[c], sem
                )
            )
        for cp in copies:
            cp.wait()
        pltpu.sync_copy(rows_v, out_hbm.at[wid])

    return gather_kernel, n_workers, n_chunks


def kernel(user_id, item_id, user_table, item_table):
    batch = user_id.shape[0]
    emb = user_table.shape[1]
    info = plsc.get_sparse_core_info()
    fn, n_workers, n_chunks = _build(
        batch, emb, info.num_cores, info.num_subcores
    )

    uid = user_id.astype(jnp.int32).reshape(n_workers, n_chunks, _CHUNK)
    iid = item_id.astype(jnp.int32).reshape(n_workers, n_chunks, _CHUNK)

    u_rows = fn(uid, user_table.astype(jnp.bfloat16))
    i_rows = fn(iid, item_table.astype(jnp.bfloat16))
    return (
        u_rows.reshape(batch, emb).astype(jnp.float32),
        i_rows.reshape(batch, emb).astype(jnp.float32),
    )


# speedup vs baseline: 1.3122x; 1.3122x over previous
import functools

import jax
import jax.numpy as jnp
from jax import lax
from jax.experimental import pallas as pl
from jax.experimental.pallas import tpu as pltpu
from jax.experimental.pallas import tpu_sc as plsc

_CHUNK = 128
_LANES = 16


@functools.lru_cache(maxsize=None)
def _build(batch, emb, n_cores, n_subcores):
    n_workers = n_cores * n_subcores
    b_per_w = batch // n_workers
    n_chunks = b_per_w // _CHUNK
    n_groups = b_per_w // _LANES
    groups_per_chunk = _CHUNK // _LANES

    mesh = plsc.VectorSubcoreMesh(
        core_axis_name="c",
        subcore_axis_name="s",
        num_cores=n_cores,
        num_subcores=n_subcores,
    )

    @functools.partial(
        pl.kernel,
        mesh=mesh,
        out_type=jax.ShapeDtypeStruct(
            (n_workers, n_chunks, _CHUNK, 2 * emb), jnp.float32
        ),
        compiler_params=pltpu.CompilerParams(
            use_tc_tiling_on_sc=True, needs_layout_passes=False
        ),
        scratch_types=[
            pltpu.VMEM((n_chunks, _CHUNK), jnp.int32),
            pltpu.VMEM((n_chunks, _CHUNK), jnp.int32),
            pltpu.VMEM((n_chunks, _CHUNK, 2 * emb), jnp.float32),
            pltpu.SemaphoreType.DMA,
        ],
    )
    def gather_kernel(id_hbm, tab_hbm, out_hbm, idx_v, gidx_v, pairs_v, sem):
        wid = lax.axis_index("s") * n_cores + lax.axis_index("c")

        pltpu.sync_copy(id_hbm.at[wid], idx_v)

        def shift(g, _):
            c = g // groups_per_chunk
            r0 = (g % groups_per_chunk) * _LANES
            ids = idx_v[c, pl.ds(r0, _LANES)]
            gidx_v[c, pl.ds(r0, _LANES)] = lax.shift_right_logical(ids, 1)
            return ()

        lax.fori_loop(0, n_groups, shift, ())

        copies = []
        for c in range(n_chunks):
            copies.append(
                pltpu.async_copy(tab_hbm.at[gidx_v.at[c]], pairs_v.at[c], sem)
            )
        for cp in copies:
            cp.wait()

        lane_iota = lax.iota(jnp.int32, _LANES)

        def select(g, _):
            c = g // groups_per_chunk
            r0 = (g % groups_per_chunk) * _LANES
            ids = idx_v[c, pl.ds(r0, _LANES)]
            odd = (ids & 1) == 1
            c_vec = jnp.full((_LANES,), c, jnp.int32)
            row_vec = r0 + lane_iota
            for d in range(emb):
                vals = plsc.load_gather(
                    pairs_v,
                    [c_vec, row_vec, jnp.full((_LANES,), emb + d, jnp.int32)],
                    mask=odd,
                )
                plsc.store_scatter(
                    pairs_v,
                    [c_vec, row_vec, jnp.full((_LANES,), d, jnp.int32)],
                    vals,
                    mask=odd,
                )
            return ()

        lax.fori_loop(0, n_groups, select, ())

        pltpu.sync_copy(pairs_v, out_hbm.at[wid])

    return gather_kernel, n_workers, n_chunks


def kernel(user_id, item_id, user_table, item_table):
    batch = user_id.shape[0]
    n_rows, emb = user_table.shape
    info = plsc.get_sparse_core_info()
    fn, n_workers, n_chunks = _build(
        batch, emb, info.num_cores, info.num_subcores
    )

    uid = user_id.astype(jnp.int32).reshape(n_workers, n_chunks, _CHUNK)
    iid = item_id.astype(jnp.int32).reshape(n_workers, n_chunks, _CHUNK)

    utab = user_table.reshape(n_rows // 2, 2 * emb)
    itab = item_table.reshape(n_rows // 2, 2 * emb)

    u_rows = fn(uid, utab)
    i_rows = fn(iid, itab)
    return (
        u_rows.reshape(batch, 2 * emb)[:, :emb],
        i_rows.reshape(batch, 2 * emb)[:, :emb],
    )


# speedup vs baseline: 3.7117x; 2.8287x over previous
import functools

import jax
import jax.numpy as jnp
from jax import lax
from jax.experimental import pallas as pl
from jax.experimental.pallas import tpu as pltpu
from jax.experimental.pallas import tpu_sc as plsc

_LANES = 16
_CW = 256
_RB = 128


@functools.lru_cache(maxsize=None)
def _build(batch, emb, n_rows, n_cores, n_subcores):
    n_full = n_rows // _CW
    tail_lo = n_full * _CW
    tail_w = n_rows - tail_lo
    max_chunks = -(-n_full // n_subcores)
    out_rows = batch + _RB
    trash_lo = batch
    groups_all = batch // _LANES

    mesh = plsc.VectorSubcoreMesh(
        core_axis_name="c",
        subcore_axis_name="s",
        num_cores=n_cores,
        num_subcores=n_subcores,
    )

    wide = jax.ShapeDtypeStruct((out_rows, 2 * emb), jnp.float32)

    @functools.partial(
        pl.kernel,
        mesh=mesh,
        out_type=(wide, wide),
        compiler_params=pltpu.CompilerParams(
            use_tc_tiling_on_sc=True, needs_layout_passes=False
        ),
        scratch_types=[
            pltpu.VMEM((batch,), jnp.int32),
            pltpu.VMEM((batch,), jnp.int32),
            pltpu.VMEM((batch,), jnp.int32),
            pltpu.VMEM((batch,), jnp.int32),
            pltpu.VMEM((2, emb, _CW), jnp.float32),
            pltpu.VMEM((emb, tail_w), jnp.float32),
            pltpu.VMEM((_RB, 2 * emb), jnp.float32),
            pltpu.VMEM((_RB,), jnp.int32),
            pltpu.SemaphoreType.DMA,
            pltpu.SemaphoreType.DMA,
        ],
    )
    def sweep_kernel(
        uid_hbm,
        iid_hbm,
        utab_hbm,
        itab_hbm,
        uout_hbm,
        iout_hbm,
        ids_v,
        bpos_v,
        mj_v,
        mp_v,
        ring_v,
        tail_v,
        rows_v,
        pos_v,
        sem,
        fsem,
    ):
        core = lax.axis_index("c")
        s = lax.axis_index("s")
        iota = lax.iota(jnp.int32, _LANES)

        def reset_trash():
            for t in range(_RB // _LANES):
                pos_v[pl.ds(t * _LANES, _LANES)] = (
                    trash_lo + t * _LANES + iota
                )

        def process(ids_hbm, tab_hbm, out_hbm):
            pltpu.sync_copy(ids_hbm, ids_v)
            reset_trash()

            def bucket(t, bcnt):
                ids = ids_v[pl.ds(t * _LANES, _LANES)]
                chunk = lax.shift_right_logical(ids, 8)
                own_full = ((chunk & (n_subcores - 1)) == s) & (
                    ids < tail_lo
                )
                own = jnp.where(
                    s == n_subcores - 1, own_full | (ids >= tail_lo), own_full
                )
                mi = jnp.where(own, 1, 0)
                cs = lax.cumsum(mi)
                slots = bcnt + cs - 1
                plsc.store_scatter(ids_v, [slots], ids, mask=own)
                plsc.store_scatter(
                    bpos_v, [slots], t * _LANES + iota, mask=own
                )
                return bcnt + cs[15]

            bcnt = lax.fori_loop(0, groups_all, bucket, 0)
            bgroups = (bcnt + _LANES - 1) // _LANES

            def filter_chunk(k, lane_mod):
                def fbody(t, mcnt):
                    ids = ids_v[pl.ds(t * _LANES, _LANES)]
                    pos = bpos_v[pl.ds(t * _LANES, _LANES)]
                    valid = (t * _LANES + iota) < bcnt
                    m = (lax.shift_right_logical(ids, 8) == k) & valid
                    mi = jnp.where(m, 1, 0)
                    cs = lax.cumsum(mi)
                    slots = mcnt + cs - 1
                    plsc.store_scatter(
                        mj_v, [slots], ids & (lane_mod - 1), mask=m
                    )
                    plsc.store_scatter(mp_v, [slots], pos, mask=m)
                    return mcnt + cs[15]

                return lax.fori_loop(0, bgroups, fbody, 0)

            def extract(buf_ref, mcnt, rb):
                def gbody(g, rb):
                    def flush():
                        pltpu.async_copy(
                            rows_v, out_hbm.at[pos_v], fsem
                        ).wait()
                        reset_trash()

                    pl.when(rb > _RB - _LANES)(flush)
                    rb = jnp.where(rb > _RB - _LANES, 0, rb)

                    jv = mj_v[pl.ds(g * _LANES, _LANES)]
                    pv = mp_v[pl.ds(g * _LANES, _LANES)]
                    ok = (g * _LANES + iota) < mcnt
                    slots = rb + iota
                    plsc.store_scatter(pos_v, [slots], pv, mask=ok)
                    for d in range(emb):
                        vals = plsc.load_gather(
                            buf_ref,
                            [jnp.full((_LANES,), d, jnp.int32), jv],
                            mask=ok,
                        )
                        plsc.store_scatter(
                            rows_v,
                            [slots, jnp.full((_LANES,), d, jnp.int32)],
                            vals,
                            mask=ok,
                        )
                    okn = jnp.where(ok, 1, 0)
                    return rb + lax.cumsum(okn)[15]

                ggroups = (mcnt + _LANES - 1) // _LANES
                return lax.fori_loop(0, ggroups, gbody, rb)

            k0 = s

            @pl.when(k0 < n_full)
            def _():
                pltpu.async_copy(
                    tab_hbm.at[:, pl.ds(k0 * _CW, _CW)], ring_v.at[0], sem
                )

            def chunk_loop(j, rb):
                k = s + j * n_subcores
                jb = j & 1

                @pl.when(k < n_full)
                def _():
                    pltpu.make_async_copy(
                        tab_hbm.at[:, pl.ds(0, _CW)], ring_v.at[jb], sem
                    ).wait()

                kn = k + n_subcores

                @pl.when(kn < n_full)
                def _():
                    pltpu.async_copy(
                        tab_hbm.at[:, pl.ds(kn * _CW, _CW)],
                        ring_v.at[1 - jb],
                        sem,
                    )

                def work(rb):
                    mcnt = filter_chunk(k, _CW)
                    return extract(ring_v.at[jb], mcnt, rb)

                return jnp.where(k < n_full, work(rb), rb)

            rb = lax.fori_loop(0, max_chunks, chunk_loop, 0)

            @pl.when(s == n_subcores - 1)
            def _():
                pltpu.sync_copy(
                    tab_hbm.at[:, pl.ds(tail_lo, tail_w)], tail_v
                )

            def tail_work(rb):
                def fbody(t, mcnt):
                    ids = ids_v[pl.ds(t * _LANES, _LANES)]
                    pos = bpos_v[pl.ds(t * _LANES, _LANES)]
                    valid = (t * _LANES + iota) < bcnt
                    m = (ids >= tail_lo) & valid
                    mi = jnp.where(m, 1, 0)
                    cs = lax.cumsum(mi)
                    slots = mcnt + cs - 1
                    plsc.store_scatter(mj_v, [slots], ids - tail_lo, mask=m)
                    plsc.store_scatter(mp_v, [slots], pos, mask=m)
                    return mcnt + cs[15]

                mcnt = lax.fori_loop(0, bgroups, fbody, 0)
                return extract(tail_v, mcnt, rb)

            rb = jnp.where(s == n_subcores - 1, tail_work(rb), rb)

            @pl.when(rb > 0)
            def _():
                pltpu.async_copy(rows_v, out_hbm.at[pos_v], fsem).wait()

        @pl.when(core == 0)
        def _():
            process(uid_hbm, utab_hbm, uout_hbm)

        @pl.when(core == 1)
        def _():
            process(iid_hbm, itab_hbm, iout_hbm)

    return sweep_kernel


def kernel(user_id, item_id, user_table, item_table):
    batch = user_id.shape[0]
    n_rows, emb = user_table.shape
    info = plsc.get_sparse_core_info()
    fn = _build(batch, emb, n_rows, info.num_cores, info.num_subcores)

    uid = user_id.astype(jnp.int32)
    iid = item_id.astype(jnp.int32)

    u_wide, i_wide = fn(uid, iid, user_table.T, item_table.T)
    return (u_wide[:batch, :emb], i_wide[:batch, :emb])


# speedup vs baseline: 4.1388x; 1.1151x over previous
import functools

import jax
import jax.numpy as jnp
from jax import lax
from jax.experimental import pallas as pl
from jax.experimental.pallas import tpu as pltpu
from jax.experimental.pallas import tpu_sc as plsc

_LANES = 16
_CW = 256
_RB = 128


@functools.lru_cache(maxsize=None)
def _build(batch, emb, n_rows, n_cores, n_subcores):
    n_full = n_rows // _CW
    tail_lo = n_full * _CW
    tail_w = n_rows - tail_lo
    max_chunks = -(-n_full // n_subcores)
    out_rows = batch + _RB
    trash_lo = batch
    groups_all = batch // _LANES

    mesh = plsc.VectorSubcoreMesh(
        core_axis_name="c",
        subcore_axis_name="s",
        num_cores=n_cores,
        num_subcores=n_subcores,
    )

    wide = jax.ShapeDtypeStruct((out_rows, 2 * emb), jnp.float32)

    @functools.partial(
        pl.kernel,
        mesh=mesh,
        out_type=(wide, wide),
        compiler_params=pltpu.CompilerParams(
            use_tc_tiling_on_sc=True, needs_layout_passes=False
        ),
        scratch_types=[
            pltpu.VMEM((batch,), jnp.int32),
            pltpu.VMEM((batch,), jnp.int32),
            pltpu.VMEM((batch,), jnp.int32),
            pltpu.VMEM((batch,), jnp.int32),
            pltpu.VMEM((2, emb, _CW), jnp.float32),
            pltpu.VMEM((emb, tail_w), jnp.float32),
            pltpu.VMEM((_RB, 2 * emb), jnp.float32),
            pltpu.VMEM((_RB,), jnp.int32),
            pltpu.SemaphoreType.DMA,
            pltpu.SemaphoreType.DMA,
        ],
    )
    def sweep_kernel(
        uid_hbm,
        iid_hbm,
        utab_hbm,
        itab_hbm,
        uout_hbm,
        iout_hbm,
        ids_v,
        bpos_v,
        mj_v,
        mp_v,
        ring_v,
        tail_v,
        rows_v,
        pos_v,
        sem,
        fsem,
    ):
        core = lax.axis_index("c")
        s = lax.axis_index("s")
        iota = lax.iota(jnp.int32, _LANES)

        def reset_trash():
            for t in range(_RB // _LANES):
                pos_v[pl.ds(t * _LANES, _LANES)] = (
                    trash_lo + t * _LANES + iota
                )

        def process(ids_hbm, tab_hbm, out_hbm):
            pltpu.sync_copy(ids_hbm, ids_v)
            reset_trash()

            def bucket(t, bcnt):
                ids = ids_v[pl.ds(t * _LANES, _LANES)]
                chunk = lax.shift_right_logical(ids, 8)
                own_full = ((chunk & (n_subcores - 1)) == s) & (
                    ids < tail_lo
                )
                own = jnp.where(
                    s == n_subcores - 1, own_full | (ids >= tail_lo), own_full
                )
                mi = jnp.where(own, 1, 0)
                cs = lax.cumsum(mi)
                slots = bcnt + cs - 1
                plsc.store_scatter(ids_v, [slots], ids, mask=own)
                plsc.store_scatter(
                    bpos_v, [slots], t * _LANES + iota, mask=own
                )
                return bcnt + cs[15]

            bcnt = lax.fori_loop(0, groups_all, bucket, 0)
            bgroups = (bcnt + _LANES - 1) // _LANES

            def filter_chunk(k, lane_mod):
                def fbody(t, mcnt):
                    ids = ids_v[pl.ds(t * _LANES, _LANES)]
                    pos = bpos_v[pl.ds(t * _LANES, _LANES)]
                    valid = (t * _LANES + iota) < bcnt
                    m = (lax.shift_right_logical(ids, 8) == k) & valid
                    mi = jnp.where(m, 1, 0)
                    cs = lax.cumsum(mi)
                    slots = mcnt + cs - 1
                    plsc.store_scatter(
                        mj_v, [slots], ids & (lane_mod - 1), mask=m
                    )
                    plsc.store_scatter(mp_v, [slots], pos, mask=m)
                    return mcnt + cs[15]

                return lax.fori_loop(0, bgroups, fbody, 0)

            def extract(buf_ref, mcnt, rb):
                def gbody(g, rb):
                    def flush():
                        pltpu.async_copy(
                            rows_v, out_hbm.at[pos_v], fsem
                        ).wait()
                        reset_trash()

                    pl.when(rb > _RB - _LANES)(flush)
                    rb = jnp.where(rb > _RB - _LANES, 0, rb)

                    jv = mj_v[pl.ds(g * _LANES, _LANES)]
                    pv = mp_v[pl.ds(g * _LANES, _LANES)]
                    ok = (g * _LANES + iota) < mcnt
                    slots = rb + iota
                    plsc.store_scatter(pos_v, [slots], pv, mask=ok)
                    for d in range(emb):
                        vals = plsc.load_gather(
                            buf_ref,
                            [jnp.full((_LANES,), d, jnp.int32), jv],
                            mask=ok,
                        )
                        plsc.store_scatter(
                            rows_v,
                            [slots, jnp.full((_LANES,), d, jnp.int32)],
                            vals,
                            mask=ok,
                        )
                    okn = jnp.where(ok, 1, 0)
                    return rb + lax.cumsum(okn)[15]

                ggroups = (mcnt + _LANES - 1) // _LANES
                return lax.fori_loop(0, ggroups, gbody, rb)

            k0 = s

            @pl.when(k0 < n_full)
            def _():
                pltpu.async_copy(
                    tab_hbm.at[:, pl.ds(k0 * _CW, _CW)], ring_v.at[0], sem
                )

            def chunk_loop(j, rb):
                k = s + j * n_subcores
                jb = j & 1
                kn = k + n_subcores

                @pl.when(kn < n_full)
                def _():
                    pltpu.async_copy(
                        tab_hbm.at[:, pl.ds(kn * _CW, _CW)],
                        ring_v.at[1 - jb],
                        sem,
                    )

                mcnt = filter_chunk(k, _CW)

                @pl.when(k < n_full)
                def _():
                    pltpu.make_async_copy(
                        tab_hbm.at[:, pl.ds(0, _CW)], ring_v.at[jb], sem
                    ).wait()

                return extract(ring_v.at[jb], mcnt, rb)

            rb = lax.fori_loop(0, max_chunks, chunk_loop, 0)

            @pl.when(s == n_subcores - 1)
            def _():
                pltpu.sync_copy(
                    tab_hbm.at[:, pl.ds(tail_lo, tail_w)], tail_v
                )

            def tail_work(rb):
                def fbody(t, mcnt):
                    ids = ids_v[pl.ds(t * _LANES, _LANES)]
                    pos = bpos_v[pl.ds(t * _LANES, _LANES)]
                    valid = (t * _LANES + iota) < bcnt
                    m = (ids >= tail_lo) & valid
                    mi = jnp.where(m, 1, 0)
                    cs = lax.cumsum(mi)
                    slots = mcnt + cs - 1
                    plsc.store_scatter(mj_v, [slots], ids - tail_lo, mask=m)
                    plsc.store_scatter(mp_v, [slots], pos, mask=m)
                    return mcnt + cs[15]

                mcnt = lax.fori_loop(0, bgroups, fbody, 0)
                return extract(tail_v, mcnt, rb)

            rb = jnp.where(s == n_subcores - 1, tail_work(rb), rb)

            @pl.when(rb > 0)
            def _():
                pltpu.async_copy(rows_v, out_hbm.at[pos_v], fsem).wait()

        @pl.when(core == 0)
        def _():
            process(uid_hbm, utab_hbm, uout_hbm)

        @pl.when(core == 1)
        def _():
            process(iid_hbm, itab_hbm, iout_hbm)

    return sweep_kernel


def kernel(user_id, item_id, user_table, item_table):
    batch = user_id.shape[0]
    n_rows, emb = user_table.shape
    info = plsc.get_sparse_core_info()
    fn = _build(batch, emb, n_rows, info.num_cores, info.num_subcores)

    uid = user_id.astype(jnp.int32)
    iid = item_id.astype(jnp.int32)

    u_wide, i_wide = fn(uid, iid, user_table.T, item_table.T)
    return (u_wide[:batch, :emb], i_wide[:batch, :emb])
